# baseline (speedup 1.0000x reference)
import jax
import jax.numpy as jnp
from jax.experimental import pallas as pl
from jax.experimental.pallas import tpu as pltpu

_N_ITEMS = 1_000_000
_D = 64
_BLOCK = 8_000
_GRID = _N_ITEMS // _BLOCK
_K = 100


def _fused_body(uid_ref, user_ref, item_ref, out_ref, sbuf, rmbuf):
    i = pl.program_id(0)
    neg_inf = jnp.float32(-jnp.inf)
    big = jnp.int32(2**30)
    iota_seg = jax.lax.broadcasted_iota(jnp.int32, (_GRID, 1), 0)
    iota_lane = jax.lax.broadcasted_iota(jnp.int32, (1, _BLOCK), 1)
    iota_out = jax.lax.broadcasted_iota(jnp.int32, (1, 128), 1)

    @pl.when(i < _GRID)
    def _score():
        u = user_ref[0, :, :]
        item_r = item_ref[...].astype(jnp.bfloat16).astype(jnp.float32)
        sc = jax.lax.dot_general(
            u, item_r,
            dimension_numbers=(((1,), (1,)), ((), ())),
            preferred_element_type=jnp.float32,
        )
        sbuf[pl.ds(i, 1), :] = sc
        rmbuf[pl.ds(i, 1), :] = jnp.max(sc).reshape(1, 1)

    @pl.when(i == _GRID)
    def _select():
        rm0 = rmbuf[0:_GRID, :]

        def body(t, carry):
            rm, out_row = carry
            m = jnp.max(rm)
            seg = jnp.min(jnp.where(rm == m, iota_seg, big))
            row = sbuf[pl.ds(seg, 1), :]
            lane = jnp.min(jnp.where(row == m, iota_lane, big))
            idx = seg * _BLOCK + lane
            newrow = jnp.where(iota_lane == lane, neg_inf, row)
            sbuf[pl.ds(seg, 1), :] = newrow
            rm = jnp.where(iota_seg == seg, jnp.max(newrow), rm)
            out_row = jnp.where(iota_out == t, idx, out_row)
            return rm, out_row

        _, out_row = jax.lax.fori_loop(
            0, _K, body, (rm0, jnp.zeros((1, 128), jnp.int32)))
        out_ref[...] = out_row


def kernel(user_id, user_emb, item_emb, topk):
    uid = jnp.asarray(user_id, dtype=jnp.int32).reshape((1,))
    user3 = (user_emb.astype(jnp.bfloat16).astype(jnp.float32)
             .reshape((user_emb.shape[0], 1, _D)))
    grid_spec = pltpu.PrefetchScalarGridSpec(
        num_scalar_prefetch=1,
        grid=(_GRID + 1,),
        in_specs=[
            pl.BlockSpec((1, 1, _D), lambda i, uid_ref: (uid_ref[0], 0, 0)),
            pl.BlockSpec(
                (_BLOCK, _D),
                lambda i, uid_ref: (jnp.minimum(i, _GRID - 1), 0)),
        ],
        out_specs=pl.BlockSpec((1, 128), lambda i, uid_ref: (0, 0)),
        scratch_shapes=[
            pltpu.MemorySpace.VMEM((_GRID, _BLOCK), jnp.float32),
            pltpu.MemorySpace.VMEM((_GRID, 1), jnp.float32),
        ],
    )
    out = pl.pallas_call(
        _fused_body,
        grid_spec=grid_spec,
        out_shape=jax.ShapeDtypeStruct((1, 128), jnp.int32),
    )(uid, user3, item_emb)
    return out[0, :_K]


# speedup vs baseline: 1.1179x; 1.1179x over previous
import jax
import jax.numpy as jnp
from jax.experimental import pallas as pl
from jax.experimental.pallas import tpu as pltpu

_N_ITEMS = 1_000_000
_D = 64
_BLOCK = 10_000
_GRID = _N_ITEMS // _BLOCK
_NBUF = 8
_K = 100


def _fused_body(uid_ref, user_ref, item_hbm, out_ref, sbuf, buf, sem):
    u = user_ref[0, :, :]
    neg_inf = jnp.float32(-jnp.inf)
    big = jnp.int32(2**30)
    iota_seg = jax.lax.broadcasted_iota(jnp.int32, (_GRID, 1), 0)
    iota_lane = jax.lax.broadcasted_iota(jnp.int32, (1, _BLOCK), 1)
    iota_out = jax.lax.broadcasted_iota(jnp.int32, (1, 128), 1)

    def _copy(i, b):
        return pltpu.make_async_copy(
            item_hbm.at[pl.ds(i * _BLOCK, _BLOCK), :],
            buf.at[pl.ds(b * _BLOCK, _BLOCK), :],
            sem.at[b])

    for b in range(_NBUF):
        _copy(b, b).start()

    def step(i, rm):
        b = jax.lax.rem(i, _NBUF)
        _copy(i, b).wait()
        raw = buf[pl.ds(b * _BLOCK, _BLOCK), :]
        item_r = raw.astype(jnp.bfloat16).astype(jnp.float32)
        sc = jax.lax.dot_general(
            u, item_r,
            dimension_numbers=(((1,), (1,)), ((), ())),
            preferred_element_type=jnp.float32,
        )
        sbuf[pl.ds(i, 1), :] = sc
        rm = jnp.where(iota_seg == i, jnp.max(sc), rm)

        @pl.when(i + _NBUF < _GRID)
        def _():
            _copy(i + _NBUF, b).start()
        return rm

    rm0 = jax.lax.fori_loop(
        0, _GRID, step, jnp.full((_GRID, 1), neg_inf, jnp.float32))

    def body(t, carry):
        rm, out_row = carry
        m = jnp.max(rm)
        seg = jnp.min(jnp.where(rm == m, iota_seg, big))
        row = sbuf[pl.ds(seg, 1), :]
        lane = jnp.min(jnp.where(row == m, iota_lane, big))
        idx = seg * _BLOCK + lane
        newrow = jnp.where(iota_lane == lane, neg_inf, row)
        sbuf[pl.ds(seg, 1), :] = newrow
        rm = jnp.where(iota_seg == seg, jnp.max(newrow), rm)
        out_row = jnp.where(iota_out == t, idx, out_row)
        return rm, out_row

    _, out_row = jax.lax.fori_loop(
        0, _K, body, (rm0, jnp.zeros((1, 128), jnp.int32)))
    out_ref[...] = out_row


def kernel(user_id, user_emb, item_emb, topk):
    uid = jnp.asarray(user_id, dtype=jnp.int32).reshape((1,))
    user3 = (user_emb.astype(jnp.bfloat16).astype(jnp.float32)
             .reshape((user_emb.shape[0], 1, _D)))
    grid_spec = pltpu.PrefetchScalarGridSpec(
        num_scalar_prefetch=1,
        grid=(1,),
        in_specs=[
            pl.BlockSpec((1, 1, _D), lambda i, uid_ref: (uid_ref[0], 0, 0)),
            pl.BlockSpec(memory_space=pltpu.MemorySpace.HBM),
        ],
        out_specs=pl.BlockSpec((1, 128), lambda i, uid_ref: (0, 0)),
        scratch_shapes=[
            pltpu.MemorySpace.VMEM((_GRID, _BLOCK), jnp.float32),
            pltpu.MemorySpace.VMEM((_NBUF * _BLOCK, _D), jnp.float32),
            pltpu.SemaphoreType.DMA((_NBUF,)),
        ],
    )
    out = pl.pallas_call(
        _fused_body,
        grid_spec=grid_spec,
        out_shape=jax.ShapeDtypeStruct((1, 128), jnp.int32),
    )(uid, user3, item_emb)
    return out[0, :_K]


# speedup vs baseline: 1.1216x; 1.0033x over previous
import jax
import jax.numpy as jnp
from jax.experimental import pallas as pl
from jax.experimental.pallas import tpu as pltpu

_N_ITEMS = 1_000_000
_D = 64
_BLOCK = 8_000
_GRID = _N_ITEMS // _BLOCK
_NBUF = 8
_K = 100


def _fused_body(uid_ref, user_ref, item_hbm, out_ref, sbuf, buf, sem):
    u = user_ref[0, :, :]
    neg_inf = jnp.float32(-jnp.inf)
    big = jnp.int32(2**30)
    iota_seg = jax.lax.broadcasted_iota(jnp.int32, (_GRID, 1), 0)
    iota_lane = jax.lax.broadcasted_iota(jnp.int32, (1, _BLOCK), 1)
    iota_out = jax.lax.broadcasted_iota(jnp.int32, (1, 128), 1)

    def _copy(i, b):
        return pltpu.make_async_copy(
            item_hbm.at[pl.ds(i * _BLOCK, _BLOCK), :],
            buf.at[pl.ds(b * _BLOCK, _BLOCK), :],
            sem.at[b])

    for b in range(_NBUF):
        _copy(b, b).start()

    def step(i, rm):
        b = jax.lax.rem(i, _NBUF)
        _copy(i, b).wait()
        raw = buf[pl.ds(b * _BLOCK, _BLOCK), :]
        item_r = raw.astype(jnp.bfloat16).astype(jnp.float32)
        sc = jax.lax.dot_general(
            u, item_r,
            dimension_numbers=(((1,), (1,)), ((), ())),
            preferred_element_type=jnp.float32,
        )
        sbuf[pl.ds(i, 1), :] = sc
        rm = jnp.where(iota_seg == i, jnp.max(sc), rm)

        @pl.when(i + _NBUF < _GRID)
        def _():
            _copy(i + _NBUF, b).start()
        return rm

    rm0 = jax.lax.fori_loop(
        0, _GRID, step, jnp.full((_GRID, 1), neg_inf, jnp.float32))

    def body(t, carry):
        rm, out_row = carry
        m = jnp.max(rm)
        seg = jnp.min(jnp.where(rm == m, iota_seg, big))
        row = sbuf[pl.ds(seg, 1), :]
        lane = jnp.min(jnp.where(row == m, iota_lane, big))
        idx = seg * _BLOCK + lane
        newrow = jnp.where(iota_lane == lane, neg_inf, row)
        sbuf[pl.ds(seg, 1), :] = newrow
        rm = jnp.where(iota_seg == seg, jnp.max(newrow), rm)
        out_row = jnp.where(iota_out == t, idx, out_row)
        return rm, out_row

    _, out_row = jax.lax.fori_loop(
        0, _K, body, (rm0, jnp.zeros((1, 128), jnp.int32)))
    out_ref[...] = out_row


def kernel(user_id, user_emb, item_emb, topk):
    uid = jnp.asarray(user_id, dtype=jnp.int32).reshape((1,))
    user3 = (user_emb.astype(jnp.bfloat16).astype(jnp.float32)
             .reshape((user_emb.shape[0], 1, _D)))
    grid_spec = pltpu.PrefetchScalarGridSpec(
        num_scalar_prefetch=1,
        grid=(1,),
        in_specs=[
            pl.BlockSpec((1, 1, _D), lambda i, uid_ref: (uid_ref[0], 0, 0)),
            pl.BlockSpec(memory_space=pltpu.MemorySpace.HBM),
        ],
        out_specs=pl.BlockSpec((1, 128), lambda i, uid_ref: (0, 0)),
        scratch_shapes=[
            pltpu.MemorySpace.VMEM((_GRID, _BLOCK), jnp.float32),
            pltpu.MemorySpace.VMEM((_NBUF * _BLOCK, _D), jnp.float32),
            pltpu.SemaphoreType.DMA((_NBUF,)),
        ],
    )
    out = pl.pallas_call(
        _fused_body,
        grid_spec=grid_spec,
        out_shape=jax.ShapeDtypeStruct((1, 128), jnp.int32),
    )(uid, user3, item_emb)
    return out[0, :_K]
